# baseline (speedup 1.0000x reference)
import jax
import jax.numpy as jnp
from jax.experimental import pallas as pl

B = 4
L = 2048
D_MODEL = 1024
MAX_DIST = 20
TBL = 32
TL = 2048


def _body(dist_ref, table_ref, out_ref):
    d = dist_ref[...]
    s = jnp.sum(d, axis=1)
    idx = jnp.clip(s // L, 0, MAX_DIST)
    cols = jax.lax.broadcasted_iota(jnp.int32, (TL, TBL), 1)
    oh = ((idx[:, None] == cols) & (cols > 0)).astype(jnp.float32)
    out_ref[...] = jnp.dot(
        oh,
        table_ref[...],
        preferred_element_type=jnp.float32,
        precision=jax.lax.Precision.HIGHEST,
    )


def kernel(dist_matrix, embed):
    n_rows = B * L
    dist2 = dist_matrix.reshape(n_rows, L)
    table = jnp.zeros((TBL, D_MODEL), jnp.float32).at[: MAX_DIST + 2].set(embed)

    out = pl.pallas_call(
        _body,
        grid=(n_rows // TL,),
        in_specs=[
            pl.BlockSpec((TL, L), lambda i: (i, 0)),
            pl.BlockSpec((TBL, D_MODEL), lambda i: (0, 0)),
        ],
        out_specs=pl.BlockSpec((TL, D_MODEL), lambda i: (i, 0)),
        out_shape=jax.ShapeDtypeStruct((n_rows, D_MODEL), jnp.float32),
    )(dist2, table)
    return out.reshape(B, L, D_MODEL)


# speedup vs baseline: 1.0181x; 1.0181x over previous
import jax
import jax.numpy as jnp
from jax.experimental import pallas as pl

B = 4
L = 2048
D_MODEL = 1024
MAX_DIST = 20
TBL = 32
TL = 1024


def _body(dist_ref, table_ref, out_ref):
    d = dist_ref[...]
    s = jnp.sum(d, axis=1)
    idx = jnp.clip(s // L, 0, MAX_DIST)
    cols = jax.lax.broadcasted_iota(jnp.int32, (TL, TBL), 1)
    oh = ((idx[:, None] == cols) & (cols > 0)).astype(jnp.float32)
    out_ref[...] = jnp.dot(
        oh,
        table_ref[...],
        preferred_element_type=jnp.float32,
        precision=jax.lax.Precision.HIGHEST,
    )


def kernel(dist_matrix, embed):
    n_rows = B * L
    dist2 = dist_matrix.reshape(n_rows, L)
    table = jnp.zeros((TBL, D_MODEL), jnp.float32).at[: MAX_DIST + 2].set(embed)

    out = pl.pallas_call(
        _body,
        grid=(n_rows // TL,),
        in_specs=[
            pl.BlockSpec((TL, L), lambda i: (i, 0)),
            pl.BlockSpec((TBL, D_MODEL), lambda i: (0, 0)),
        ],
        out_specs=pl.BlockSpec((TL, D_MODEL), lambda i: (i, 0)),
        out_shape=jax.ShapeDtypeStruct((n_rows, D_MODEL), jnp.float32),
    )(dist2, table)
    return out.reshape(B, L, D_MODEL)
